# baseline (speedup 1.0000x reference)
import functools
import numpy as np
import jax
import jax.numpy as jnp
from jax.experimental import pallas as pl
from jax.experimental.pallas import tpu as pltpu

_R = 2048
_TRIU_I, _TRIU_J = np.triu_indices(8, k=1)


def _build_tables(s_ref, i3_ref, i4_ref, i5_ref, a_ref, b_ref, c3_ref,
                  c4_ref, c5_ref, e_scr, g_scr):
    d128 = jax.lax.broadcasted_iota(jnp.int32, (128, 16), 0)
    c16 = jax.lax.broadcasted_iota(jnp.int32, (128, 16), 1)
    for j in range(8):
        lj = s_ref[j] % 128
        e_scr[j] = ((d128 == lj) & (c16 == j)).astype(jnp.float32)

    g_scr[...] = jnp.zeros((16, 640), jnp.float32)
    row16 = jax.lax.broadcasted_iota(jnp.int32, (16, 1), 0)

    def put(k, t, f, scale):
        col = (row16 == f).astype(jnp.float32)
        if scale is not None:
            col = col * scale
        g_scr[:, pl.ds(k * 128 + t, 1)] = col

    for t in range(60):
        if t < 8:
            deg, comps = 1, [t]
            sc = a_ref[t]
        elif t < 36:
            deg, comps = 2, [int(_TRIU_I[t - 8]), int(_TRIU_J[t - 8])]
            sc = b_ref[comps[0], comps[1]]
        elif t < 48:
            m = t - 36
            deg, comps = 3, [i3_ref[m, 0], i3_ref[m, 1], i3_ref[m, 2]]
            sc = c3_ref[m]
        elif t < 56:
            m = t - 48
            deg = 4
            comps = [i4_ref[m, 0], i4_ref[m, 1], i4_ref[m, 2], i4_ref[m, 3]]
            sc = c4_ref[m]
        else:
            m = t - 56
            deg = 5
            comps = [i5_ref[m, 0], i5_ref[m, 1], i5_ref[m, 2], i5_ref[m, 3],
                     i5_ref[m, 4]]
            sc = c5_ref[m]
        for k in range(5):
            f = comps[k] if k < deg else 8
            put(k, t, f, sc if k == 0 else None)


def _poly_body(rb, *refs):
    srefs = refs[:9]
    xblks = refs[9:17]
    out_ref = refs[17]
    e_scr, g_scr = refs[18], refs[19]
    i = pl.program_id(0)

    @pl.when(i == 0)
    def _():
        _build_tables(*srefs, e_scr, g_scr)

    xsa = jnp.zeros((rb, 16), dtype=jnp.float32)
    for j in range(8):
        xsa = xsa + jnp.dot(xblks[j][...], e_scr[j],
                            preferred_element_type=jnp.float32)
    lane16 = jax.lax.broadcasted_iota(jnp.int32, (rb, 16), 1)
    xsa = xsa + jnp.where(lane16 == 8, 1.0, 0.0)
    v = jnp.dot(xsa, g_scr[...], preferred_element_type=jnp.float32)
    p = (v[:, 0:128] * v[:, 128:256] * v[:, 256:384]
         * v[:, 384:512] * v[:, 512:640])
    out_ref[...] = jnp.sum(p, axis=1, keepdims=True)


def kernel(x, a, b, c3, c4, c5, S, idx3, idx4, idx5):
    B, D = x.shape
    s32 = S.astype(jnp.int32)
    i3 = idx3.astype(jnp.int32)
    i4 = idx4.astype(jnp.int32)
    i5 = idx5.astype(jnp.int32)

    rb = min(_R, B)
    nb = B // rb
    grid_spec = pltpu.PrefetchScalarGridSpec(
        num_scalar_prefetch=9,
        grid=(nb,),
        in_specs=[pl.BlockSpec((rb, 128),
                               (lambda i, s, *_, j=j: (i, s[j] // 128)))
                  for j in range(8)],
        out_specs=pl.BlockSpec((rb, 1), lambda i, *_: (i, 0)),
        scratch_shapes=[
            pltpu.VMEM((8, 128, 16), jnp.float32),
            pltpu.VMEM((16, 640), jnp.float32),
        ],
    )
    out = pl.pallas_call(
        functools.partial(_poly_body, rb),
        grid_spec=grid_spec,
        out_shape=jax.ShapeDtypeStruct((B, 1), jnp.float32),
        compiler_params=pltpu.CompilerParams(
            dimension_semantics=("arbitrary",),
        ),
    )(s32, i3, i4, i5, a, b, c3, c4, c5, x, x, x, x, x, x, x, x)
    return out.reshape(B)


# speedup vs baseline: 1.0025x; 1.0025x over previous
import functools
import numpy as np
import jax
import jax.numpy as jnp
from jax.experimental import pallas as pl
from jax.experimental.pallas import tpu as pltpu

_R = 4096
_TRIU_I, _TRIU_J = np.triu_indices(8, k=1)


def _build_tables(s_ref, i3_ref, i4_ref, i5_ref, a_ref, b_ref, c3_ref,
                  c4_ref, c5_ref, e_scr, g_scr):
    d128 = jax.lax.broadcasted_iota(jnp.int32, (128, 16), 0)
    c16 = jax.lax.broadcasted_iota(jnp.int32, (128, 16), 1)
    for j in range(8):
        lj = s_ref[j] % 128
        e_scr[j] = ((d128 == lj) & (c16 == j)).astype(jnp.float32)

    g_scr[...] = jnp.zeros((16, 640), jnp.float32)
    row16 = jax.lax.broadcasted_iota(jnp.int32, (16, 1), 0)

    def put(k, t, f, scale):
        col = (row16 == f).astype(jnp.float32)
        if scale is not None:
            col = col * scale
        g_scr[:, pl.ds(k * 128 + t, 1)] = col

    for t in range(60):
        if t < 8:
            deg, comps = 1, [t]
            sc = a_ref[t]
        elif t < 36:
            deg, comps = 2, [int(_TRIU_I[t - 8]), int(_TRIU_J[t - 8])]
            sc = b_ref[comps[0], comps[1]]
        elif t < 48:
            m = t - 36
            deg, comps = 3, [i3_ref[m, 0], i3_ref[m, 1], i3_ref[m, 2]]
            sc = c3_ref[m]
        elif t < 56:
            m = t - 48
            deg = 4
            comps = [i4_ref[m, 0], i4_ref[m, 1], i4_ref[m, 2], i4_ref[m, 3]]
            sc = c4_ref[m]
        else:
            m = t - 56
            deg = 5
            comps = [i5_ref[m, 0], i5_ref[m, 1], i5_ref[m, 2], i5_ref[m, 3],
                     i5_ref[m, 4]]
            sc = c5_ref[m]
        for k in range(5):
            f = comps[k] if k < deg else 8
            put(k, t, f, sc if k == 0 else None)


def _poly_body(rb, *refs):
    srefs = refs[:9]
    xblks = refs[9:17]
    out_ref = refs[17]
    e_scr, g_scr = refs[18], refs[19]
    i = pl.program_id(0)

    @pl.when(i == 0)
    def _():
        _build_tables(*srefs, e_scr, g_scr)

    xsa = jnp.zeros((rb, 16), dtype=jnp.float32)
    for j in range(8):
        xsa = xsa + jnp.dot(xblks[j][...], e_scr[j],
                            preferred_element_type=jnp.float32)
    lane16 = jax.lax.broadcasted_iota(jnp.int32, (rb, 16), 1)
    xsa = xsa + jnp.where(lane16 == 8, 1.0, 0.0)
    v = jnp.dot(xsa, g_scr[...], preferred_element_type=jnp.float32)
    p = (v[:, 0:128] * v[:, 128:256] * v[:, 256:384]
         * v[:, 384:512] * v[:, 512:640])
    out_ref[...] = jnp.sum(p, axis=1, keepdims=True)


def kernel(x, a, b, c3, c4, c5, S, idx3, idx4, idx5):
    B, D = x.shape
    s32 = S.astype(jnp.int32)
    i3 = idx3.astype(jnp.int32)
    i4 = idx4.astype(jnp.int32)
    i5 = idx5.astype(jnp.int32)

    rb = min(_R, B)
    nb = B // rb
    grid_spec = pltpu.PrefetchScalarGridSpec(
        num_scalar_prefetch=9,
        grid=(nb,),
        in_specs=[pl.BlockSpec((rb, 128),
                               (lambda i, s, *_, j=j: (i, s[j] // 128)))
                  for j in range(8)],
        out_specs=pl.BlockSpec((rb, 1), lambda i, *_: (i, 0)),
        scratch_shapes=[
            pltpu.VMEM((8, 128, 16), jnp.float32),
            pltpu.VMEM((16, 640), jnp.float32),
        ],
    )
    out = pl.pallas_call(
        functools.partial(_poly_body, rb),
        grid_spec=grid_spec,
        out_shape=jax.ShapeDtypeStruct((B, 1), jnp.float32),
        compiler_params=pltpu.CompilerParams(
            dimension_semantics=("arbitrary",),
        ),
    )(s32, i3, i4, i5, a, b, c3, c4, c5, x, x, x, x, x, x, x, x)
    return out.reshape(B)
